# baseline (speedup 1.0000x reference)
import functools

import jax
import jax.numpy as jnp
from jax import lax
from jax.experimental import pallas as pl
from jax.experimental.pallas import tpu as pltpu
from jax.experimental.pallas import tpu_sc as plsc

N = 10000
E = 160000
DF = 128
DE = 16
G = 256
L = 128
H = 512

N_PAD = 10240
EC = E // 128
EC_PAD = 1280
NPC = N_PAD // 128

BE = 3200
BN = 2048

NUM_SC = 2
NUM_SUBCORES = 16
NW = NUM_SC * NUM_SUBCORES

@functools.lru_cache(maxsize=None)
def _sc_mesh():
    return plsc.VectorSubcoreMesh(
        core_axis_name="c", subcore_axis_name="s",
        num_cores=NUM_SC, num_subcores=NUM_SUBCORES)


def _ln(o, g, b):
    mu = jnp.mean(o, axis=-1, keepdims=True)
    var = jnp.mean((o - mu) * (o - mu), axis=-1, keepdims=True)
    return (o - mu) / jnp.sqrt(var + 1e-5) * g + b


def _enc_x_body(x_ref, w_ref, b_ref, o_ref):
    i = pl.program_id(0)
    rows = i * BN + lax.broadcasted_iota(jnp.int32, (BN, 1), 0)
    h = jnp.dot(x_ref[...], w_ref[...], preferred_element_type=jnp.float32)
    h = h + b_ref[...]
    o_ref[...] = jnp.where(rows < N, h, 0.0)


def _enc_x(x_pad, w, b, it=False):
    return pl.pallas_call(
        _enc_x_body,
        grid=(N_PAD // BN,),
        in_specs=[
            pl.BlockSpec((BN, DF), lambda i: (i, 0)),
            pl.BlockSpec((DF, L), lambda i: (0, 0)),
            pl.BlockSpec((1, L), lambda i: (0, 0)),
        ],
        out_specs=pl.BlockSpec((BN, L), lambda i: (i, 0)),
        out_shape=jax.ShapeDtypeStruct((N_PAD, L), jnp.float32),
        interpret=it,
    )(x_pad, w, b.reshape(1, L))


def _enc_e_body(a_ref, w_ref, b_ref, o_ref):
    o_ref[...] = (
        jnp.dot(a_ref[...], w_ref[...], preferred_element_type=jnp.float32)
        + b_ref[...]
    )


def _enc_e(ea, w, b, it=False):
    return pl.pallas_call(
        _enc_e_body,
        grid=(E // BE,),
        in_specs=[
            pl.BlockSpec((BE, DE), lambda i: (i, 0)),
            pl.BlockSpec((DE, L), lambda i: (0, 0)),
            pl.BlockSpec((1, L), lambda i: (0, 0)),
        ],
        out_specs=pl.BlockSpec((BE, L), lambda i: (i, 0)),
        out_shape=jax.ShapeDtypeStruct((E, L), jnp.float32),
        interpret=it,
    )(ea, w, b.reshape(1, L))


def _bdot(a, b):
    return jnp.dot(a, b, preferred_element_type=jnp.float32)


def _edge_mlp_body(e_ref, hs_ref, hd_ref, u_ref, e2g_ref, w1e_ref, w1s_ref,
                   w1d_ref, w1u_ref, b1_ref, w2_ref, b2_ref, g_ref, be_ref,
                   att_ref, en_ref, bm_ref, es_ref):
    i = pl.program_id(0)
    e = e_ref[...]
    cols = lax.broadcasted_iota(jnp.int32, (BE, G), 1)
    oh = (cols == e2g_ref[...]).astype(jnp.float32)
    ug = jnp.dot(oh, u_ref[...], preferred_element_type=jnp.float32)
    z = (
        _bdot(e, w1e_ref[...])
        + _bdot(hs_ref[...], w1s_ref[...])
        + _bdot(hd_ref[...], w1d_ref[...])
        + _bdot(ug, w1u_ref[...])
        + b1_ref[...]
    )
    z = jnp.maximum(z, 0.0)
    o = _bdot(z, w2_ref[...]) + b2_ref[...]
    e_new = e + _ln(o, g_ref[...], be_ref[...])
    en_ref[...] = e_new
    logits = jnp.sum(e_new * att_ref[...], axis=1, keepdims=True)
    bm_ref[...] = jnp.full((1, 1, L), jnp.max(logits), jnp.float32)
    es_part = lax.dot_general(oh, e_new, (((0,), (0,)), ((), ())),
                              preferred_element_type=jnp.float32)

    @pl.when(i == 0)
    def _():
        es_ref[...] = es_part

    @pl.when(i > 0)
    def _():
        es_ref[...] += es_part


def _edge_mlp(e, hs, hd, u, e2g_col, sp, att, it=False):
    w1 = sp["W1"]
    return pl.pallas_call(
        _edge_mlp_body,
        grid=(E // BE,),
        in_specs=[pl.BlockSpec((BE, L), lambda i: (i, 0))] * 3
        + [
            pl.BlockSpec((G, L), lambda i: (0, 0)),
            pl.BlockSpec((BE, 1), lambda i: (i, 0)),
        ]
        + [pl.BlockSpec((L, H), lambda i: (0, 0))] * 4
        + [
            pl.BlockSpec((1, H), lambda i: (0, 0)),
            pl.BlockSpec((H, L), lambda i: (0, 0)),
            pl.BlockSpec((1, L), lambda i: (0, 0)),
            pl.BlockSpec((1, L), lambda i: (0, 0)),
            pl.BlockSpec((1, L), lambda i: (0, 0)),
            pl.BlockSpec((1, L), lambda i: (0, 0)),
        ],
        out_specs=[
            pl.BlockSpec((BE, L), lambda i: (i, 0)),
            pl.BlockSpec((1, 1, L), lambda i: (i, 0, 0)),
            pl.BlockSpec((G, L), lambda i: (0, 0)),
        ],
        out_shape=[
            jax.ShapeDtypeStruct((E, L), jnp.float32),
            jax.ShapeDtypeStruct((E // BE, 1, L), jnp.float32),
            jax.ShapeDtypeStruct((G, L), jnp.float32),
        ],
        interpret=it,
    )(
        e, hs, hd, u, e2g_col,
        w1[0:L], w1[L:2 * L], w1[2 * L:3 * L], w1[3 * L:4 * L],
        sp["b1"].reshape(1, H), sp["W2"], sp["b2"].reshape(1, L),
        sp["g"].reshape(1, L), sp["be"].reshape(1, L), att,
    )


def _edge_exp_body(en_ref, bm_ref, att_ref, ep_ref, p_ref):
    gmax = jnp.max(bm_ref[...])
    e = en_ref[...]
    logits = jnp.sum(e * att_ref[...], axis=1, keepdims=True)
    p = jnp.exp(logits - gmax)
    ep_ref[...] = e * p
    p_ref[...] = p


def _edge_exp(e_new, bm, att, it=False):
    return pl.pallas_call(
        _edge_exp_body,
        grid=(E // BE,),
        in_specs=[
            pl.BlockSpec((BE, L), lambda i: (i, 0)),
            pl.BlockSpec((E // BE, 1, L), lambda i: (0, 0, 0)),
            pl.BlockSpec((1, L), lambda i: (0, 0)),
        ],
        out_specs=[
            pl.BlockSpec((BE, L), lambda i: (i, 0)),
            pl.BlockSpec((BE, 1), lambda i: (i, 0)),
        ],
        out_shape=[
            jax.ShapeDtypeStruct((E, L), jnp.float32),
            jax.ShapeDtypeStruct((E, 1), jnp.float32),
        ],
        interpret=it,
    )(e_new, bm, att)


def _node_mlp_body(h_ref, acc_ref, s_ref, u_ref, n2g_ref, w1h_ref, w1a_ref,
                   w1u_ref, b1_ref, w2_ref, b2_ref, g_ref, be_ref, hn_ref,
                   ns_ref):
    i = pl.program_id(0)
    rows = i * BN + lax.broadcasted_iota(jnp.int32, (BN, 1), 0)
    a = acc_ref[0] + acc_ref[1]
    agg = a / (s_ref[0] + s_ref[1] + 1e-16)
    h = h_ref[...]
    cols = lax.broadcasted_iota(jnp.int32, (BN, G), 1)
    oh = (cols == n2g_ref[...]).astype(jnp.float32)
    un = jnp.dot(oh, u_ref[...], preferred_element_type=jnp.float32)
    z = (
        _bdot(h, w1h_ref[...])
        + _bdot(agg, w1a_ref[...])
        + _bdot(un, w1u_ref[...])
        + b1_ref[...]
    )
    z = jnp.maximum(z, 0.0)
    o = _bdot(z, w2_ref[...]) + b2_ref[...]
    h_new = jnp.where(rows < N, h + _ln(o, g_ref[...], be_ref[...]), 0.0)
    hn_ref[...] = h_new
    ns_part = lax.dot_general(oh, h_new, (((0,), (0,)), ((), ())),
                              preferred_element_type=jnp.float32)

    @pl.when(i == 0)
    def _():
        ns_ref[...] = ns_part

    @pl.when(i > 0)
    def _():
        ns_ref[...] += ns_part


def _node_mlp(h, acc2, s2, u, n2g_col, sp, it=False):
    w1 = sp["W1"]
    return pl.pallas_call(
        _node_mlp_body,
        grid=(N_PAD // BN,),
        in_specs=[
            pl.BlockSpec((BN, L), lambda i: (i, 0)),
            pl.BlockSpec((2, BN, L), lambda i: (0, i, 0)),
            pl.BlockSpec((2, BN, 1), lambda i: (0, i, 0)),
            pl.BlockSpec((G, L), lambda i: (0, 0)),
            pl.BlockSpec((BN, 1), lambda i: (i, 0)),
            pl.BlockSpec((L, H), lambda i: (0, 0)),
            pl.BlockSpec((L, H), lambda i: (0, 0)),
            pl.BlockSpec((L, H), lambda i: (0, 0)),
            pl.BlockSpec((1, H), lambda i: (0, 0)),
            pl.BlockSpec((H, L), lambda i: (0, 0)),
            pl.BlockSpec((1, L), lambda i: (0, 0)),
            pl.BlockSpec((1, L), lambda i: (0, 0)),
            pl.BlockSpec((1, L), lambda i: (0, 0)),
        ],
        out_specs=[
            pl.BlockSpec((BN, L), lambda i: (i, 0)),
            pl.BlockSpec((G, L), lambda i: (0, 0)),
        ],
        out_shape=[
            jax.ShapeDtypeStruct((N_PAD, L), jnp.float32),
            jax.ShapeDtypeStruct((G, L), jnp.float32),
        ],
        interpret=it,
    )(
        h, acc2, s2, u, n2g_col,
        w1[0:L], w1[L:2 * L], w1[2 * L:3 * L],
        sp["b1"].reshape(1, H), sp["W2"], sp["b2"].reshape(1, L),
        sp["g"].reshape(1, L), sp["be"].reshape(1, L),
    )


def _seg_mm_body(rows_ref, seg_ref, out_ref, *, bm):
    i = pl.program_id(0)
    cols = lax.broadcasted_iota(jnp.int32, (bm, G), 1)
    oh = (cols == seg_ref[...]).astype(jnp.float32)
    part = lax.dot_general(oh, rows_ref[...], (((0,), (0,)), ((), ())),
                           preferred_element_type=jnp.float32)

    @pl.when(i == 0)
    def _():
        out_ref[...] = part

    @pl.when(i > 0)
    def _():
        out_ref[...] += part


def _seg_mm(rows, seg_col, bm, it=False):
    m = rows.shape[0]
    return pl.pallas_call(
        functools.partial(_seg_mm_body, bm=bm),
        grid=(m // bm,),
        in_specs=[
            pl.BlockSpec((bm, L), lambda i: (i, 0)),
            pl.BlockSpec((bm, 1), lambda i: (i, 0)),
        ],
        out_specs=pl.BlockSpec((G, L), lambda i: (0, 0)),
        out_shape=jax.ShapeDtypeStruct((G, L), jnp.float32),
        compiler_params=pltpu.CompilerParams(
            dimension_semantics=("arbitrary",)),
        interpret=it,
    )(rows, seg_col)


def _glob_mlp_body(u_ref, ns_ref, es_ref, w1u_ref, w1n_ref, w1e_ref, b1_ref,
                   w2_ref, b2_ref, g_ref, be_ref, un_ref):
    u = u_ref[...]
    ns = ns_ref[...]
    es = es_ref[...]
    z = (
        jnp.dot(u, w1u_ref[...], preferred_element_type=jnp.float32)
        + jnp.dot(ns, w1n_ref[...], preferred_element_type=jnp.float32)
        + jnp.dot(es, w1e_ref[...], preferred_element_type=jnp.float32)
        + b1_ref[...]
    )
    z = jnp.maximum(z, 0.0)
    o = jnp.dot(z, w2_ref[...], preferred_element_type=jnp.float32) + b2_ref[...]
    un_ref[...] = u + _ln(o, g_ref[...], be_ref[...])


def _glob_mlp(u, ns, es, sp, it=False):
    w1 = sp["W1"]
    return pl.pallas_call(
        _glob_mlp_body,
        grid=(1,),
        in_specs=[
            pl.BlockSpec((G, L), lambda i: (0, 0)),
            pl.BlockSpec((G, L), lambda i: (0, 0)),
            pl.BlockSpec((G, L), lambda i: (0, 0)),
            pl.BlockSpec((L, H), lambda i: (0, 0)),
            pl.BlockSpec((L, H), lambda i: (0, 0)),
            pl.BlockSpec((L, H), lambda i: (0, 0)),
            pl.BlockSpec((1, H), lambda i: (0, 0)),
            pl.BlockSpec((H, L), lambda i: (0, 0)),
            pl.BlockSpec((1, L), lambda i: (0, 0)),
            pl.BlockSpec((1, L), lambda i: (0, 0)),
            pl.BlockSpec((1, L), lambda i: (0, 0)),
        ],
        out_specs=pl.BlockSpec((G, L), lambda i: (0, 0)),
        out_shape=jax.ShapeDtypeStruct((G, L), jnp.float32),
        interpret=it,
    )(
        u, ns, es,
        w1[0:L], w1[L:2 * L], w1[2 * L:3 * L],
        sp["b1"].reshape(1, H), sp["W2"], sp["b2"].reshape(1, L),
        sp["g"].reshape(1, L), sp["be"].reshape(1, L),
    )


def _proj_body(ns_ref, u_ref, w1_ref, b1_ref, w2_ref, b2_ref, z_ref):
    mol = ns_ref[...] + u_ref[...]
    t = jnp.maximum(
        jnp.dot(mol, w1_ref[...], preferred_element_type=jnp.float32)
        + b1_ref[...], 0.0)
    z_ref[...] = (
        jnp.dot(t, w2_ref[...], preferred_element_type=jnp.float32)
        + b2_ref[...]
    )


def _proj(ns, u, pr, it=False):
    return pl.pallas_call(
        _proj_body,
        grid=(1,),
        in_specs=[
            pl.BlockSpec((G, L), lambda i: (0, 0)),
            pl.BlockSpec((G, L), lambda i: (0, 0)),
            pl.BlockSpec((L, L), lambda i: (0, 0)),
            pl.BlockSpec((1, L), lambda i: (0, 0)),
            pl.BlockSpec((L, L), lambda i: (0, 0)),
            pl.BlockSpec((1, L), lambda i: (0, 0)),
        ],
        out_specs=pl.BlockSpec((G, L), lambda i: (0, 0)),
        out_shape=jax.ShapeDtypeStruct((G, L), jnp.float32),
        interpret=it,
    )(ns, u, pr["W1"], pr["b1"].reshape(1, L), pr["W2"],
      pr["b2"].reshape(1, L))


def _worker_id():
    return lax.axis_index("s") * NUM_SC + lax.axis_index("c")


def _edge_chunk_range(w):
    c0 = w * 40
    cnt = jnp.where(w < 31, 40, EC - 31 * 40)
    return c0, cnt


def _node_chunk_range(w):
    g0 = w * 2 + jnp.minimum(w, 16)
    cnt = 2 + jnp.where(w < 16, 1, 0)
    return g0, cnt


def _stage_idx(idx_hbm, idx_v, c0):
    pltpu.sync_copy(idx_hbm.at[pl.ds(c0, 40)], idx_v)


def _sc_e2g_body(n2g_ref, dst_ref, out_ref, n2g_v, idx_v, out_v, sem):
    del sem
    w = _worker_id()
    c0, cnt = _edge_chunk_range(w)
    pltpu.sync_copy(n2g_ref, n2g_v)
    _stage_idx(dst_ref, idx_v, c0)

    @pl.loop(0, cnt)
    def _(j):
        for k in range(8):
            idx16 = idx_v[j, pl.ds(16 * k, 16)]
            out_v[j, pl.ds(16 * k, 16)] = plsc.load_gather(n2g_v, [idx16])

    pltpu.sync_copy(out_v, out_ref.at[pl.ds(c0, 40)])


def _sc_e2g(n2g, dst2d):
    return pl.kernel(
        _sc_e2g_body,
        out_type=jax.ShapeDtypeStruct((EC_PAD, 128), jnp.int32),
        mesh=_sc_mesh(),
        compiler_params=pltpu.CompilerParams(needs_layout_passes=False),
        scratch_types=[
            pltpu.VMEM((N,), jnp.int32),
            pltpu.VMEM((40, 128), jnp.int32),
            pltpu.VMEM((40, 128), jnp.int32),
            pltpu.SemaphoreType.DMA,
        ],
    )(n2g, dst2d)


def _sc_gather_body(h_ref, src_ref, dst_ref, hs_ref, hd_ref, idx_v,
                    r0, r1, r2, r3, gs0, gs1, gs2, gs3, ss0, ss1, ss2, ss3):
    w = _worker_id()
    c0, cnt = _edge_chunk_range(w)
    bufs = (r0, r1, r2, r3)
    gsems = (gs0, gs1, gs2, gs3)
    ssems = (ss0, ss1, ss2, ss3)

    for idx_hbm, table, out in (
        (src_ref, h_ref, hs_ref),
        (dst_ref, h_ref, hd_ref),
    ):
        _stage_idx(idx_hbm, idx_v, c0)

        @pl.loop(0, cnt // 4)
        def _(grp, table=table, out=out):
            j0 = 4 * grp
            gds = [
                pltpu.async_copy(table.at[idx_v.at[j0 + b]], bufs[b],
                                 gsems[b])
                for b in range(4)
            ]
            sds = []
            for b in range(4):
                gds[b].wait()
                sds.append(pltpu.async_copy(
                    bufs[b], out.at[pl.ds((c0 + j0 + b) * 128, 128)],
                    ssems[b]))
            for b in range(4):
                sds[b].wait()

        @pl.loop(4 * (cnt // 4), cnt)
        def _(j, table=table, out=out):
            pltpu.async_copy(table.at[idx_v.at[j]], r0, gs0).wait()
            pltpu.sync_copy(r0, out.at[pl.ds((c0 + j) * 128, 128)])


def _sc_gather(h_pad, src2d, dst2d):
    return pl.kernel(
        _sc_gather_body,
        out_type=[
            jax.ShapeDtypeStruct((E, L), jnp.float32),
            jax.ShapeDtypeStruct((E, L), jnp.float32),
        ],
        mesh=_sc_mesh(),
        scratch_types=[
            pltpu.VMEM((40, 128), jnp.int32),
            pltpu.VMEM((128, L), jnp.float32),
            pltpu.VMEM((128, L), jnp.float32),
            pltpu.VMEM((128, L), jnp.float32),
            pltpu.VMEM((128, L), jnp.float32),
        ] + [pltpu.SemaphoreType.DMA] * 8,
    )(h_pad, src2d, dst2d)


ACC_ROWS = N_PAD + 128
SROW = 10160


def _edge_chunk_range64(w):
    c0 = w * 80
    cnt = jnp.where(w < 31, 80, (E // 64) - 31 * 80)
    return c0, cnt


def _sc_scatter_edges_body(ep_ref, dst_ref, p_ref, zacc_ref,
                           acc_out, acc_sh, idxd_v, p_v,
                           s_local, idr_v, r0, r1, gs0, gs1, ss0, ss1):
    c = lax.axis_index("c")
    s = lax.axis_index("s")
    w = _worker_id()
    c0, cnt = _edge_chunk_range64(w)

    pltpu.sync_copy(zacc_ref.at[pl.ds(s * (ACC_ROWS // 16), ACC_ROWS // 16)],
                    acc_sh.at[pl.ds(s * (ACC_ROWS // 16), ACC_ROWS // 16)])
    pltpu.sync_copy(zacc_ref.at[pl.ds(0, NPC)], s_local)

    iota16 = lax.broadcasted_iota(jnp.int32, (16,), 0)
    for k in range(5):
        idr_v[0, pl.ds(16 * k, 16)] = iota16 + (SROW + 16 * k)

    plsc.subcore_barrier()

    pltpu.sync_copy(dst_ref.at[pl.ds(c0, 80)], idxd_v)
    pltpu.sync_copy(p_ref.at[pl.ds(c0, 80)], p_v)

    @pl.loop(0, cnt // 2)
    def _(grp):
        j0 = 2 * grp
        j1 = j0 + 1
        g0 = pltpu.async_copy(ep_ref.at[pl.ds((c0 + j0) * 64, 64)], r0, gs0)
        g1 = pltpu.async_copy(ep_ref.at[pl.ds((c0 + j1) * 64, 64)], r1, gs1)
        g0.wait()
        s0 = pltpu.async_copy(r0, acc_sh.at[idxd_v.at[j0]], ss0, add=True)
        for k in range(4):
            d16 = idxd_v[j0, pl.ds(16 * k, 16)]
            p16 = p_v[j0, pl.ds(16 * k, 16)]
            plsc.addupdate_scatter(
                s_local, [lax.shift_right_logical(d16, 7), d16 & 127], p16)
        g1.wait()
        s1 = pltpu.async_copy(r1, acc_sh.at[idxd_v.at[j1]], ss1, add=True)
        for k in range(4):
            d16 = idxd_v[j1, pl.ds(16 * k, 16)]
            p16 = p_v[j1, pl.ds(16 * k, 16)]
            plsc.addupdate_scatter(
                s_local, [lax.shift_right_logical(d16, 7), d16 & 127], p16)
        s0.wait()
        s1.wait()

    pltpu.sync_copy(s_local, acc_sh.at[idr_v.at[0]], add=True)

    plsc.subcore_barrier()

    @pl.when(s == 0)
    def _():
        pltpu.sync_copy(acc_sh, acc_out.at[c])


def _sc_scatter_edges(ep, dst64, p64, zacc):
    return pl.kernel(
        _sc_scatter_edges_body,
        out_type=jax.ShapeDtypeStruct((NUM_SC, ACC_ROWS, L), jnp.float32),
        mesh=_sc_mesh(),
        compiler_params=pltpu.CompilerParams(needs_layout_passes=False),
        scratch_types=[
            pltpu.VMEM_SHARED((ACC_ROWS, L), jnp.float32),
            pltpu.VMEM((80, 64), jnp.int32),
            pltpu.VMEM((80, 64), jnp.float32),
            pltpu.VMEM((NPC, 128), jnp.float32),
            pltpu.VMEM((1, 80), jnp.int32),
            pltpu.VMEM((64, L), jnp.float32),
            pltpu.VMEM((64, L), jnp.float32),
        ] + [pltpu.SemaphoreType.DMA] * 4,
    )(ep, dst64, p64, zacc)


def kernel(x, edge_index, edge_attr, node2graph, params):
    pad_e = EC_PAD * 128 - E
    src2d = jnp.pad(edge_index[0], (0, pad_e)).reshape(EC_PAD, 128)
    dst2d = jnp.pad(edge_index[1], (0, pad_e)).reshape(EC_PAD, 128)
    dst64 = jnp.pad(edge_index[1], (0, pad_e)).reshape(EC_PAD * 2, 64)
    n2g_col = jnp.pad(node2graph, (0, N_PAD - N)).reshape(N_PAD, 1)
    x_pad = jnp.pad(x, ((0, N_PAD - N), (0, 0)))
    zacc = jnp.zeros((ACC_ROWS, L), jnp.float32)

    e2g2d = _sc_e2g(node2graph, dst2d)
    e2g_col = e2g2d.reshape(EC_PAD * 128)[:E].reshape(E, 1)

    h = _enc_x(x_pad, params["enc_x"]["W"], params["enc_x"]["b"])
    e = _enc_e(edge_attr, params["enc_e"]["W"], params["enc_e"]["b"])
    u = jnp.zeros((G, L), jnp.float32)

    for sp in params["steps"]:
        att = sp["att"].reshape(1, L)
        hs, hd = _sc_gather(h, src2d, dst2d)
        e, bm, es = _edge_mlp(e, hs, hd, u, e2g_col, sp["edge"], att)
        ep, pcol = _edge_exp(e, bm, att)
        p64 = jnp.pad(pcol.reshape(E), (0, pad_e)).reshape(EC_PAD * 2, 64)
        acc2 = _sc_scatter_edges(ep, dst64, p64, zacc)
        s2 = acc2[:, SROW:SROW + NPC, :].reshape(NUM_SC, N_PAD)[:, :, None]
        h, ns = _node_mlp(h, acc2, s2, u, n2g_col, sp["node"])
        u = _glob_mlp(u, ns, es, sp["glob"])

    return _proj(ns, u, params["proj"])


# speedup vs baseline: 1.0487x; 1.0487x over previous
import functools

import jax
import jax.numpy as jnp
from jax import lax
from jax.experimental import pallas as pl
from jax.experimental.pallas import tpu as pltpu
from jax.experimental.pallas import tpu_sc as plsc

N = 10000
E = 160000
DF = 128
DE = 16
G = 256
L = 128
H = 512

N_PAD = 10240
EC = E // 128
EC_PAD = 1280
NPC = N_PAD // 128

BE = 3200
BN = 2048

NUM_SC = 2
NUM_SUBCORES = 16
NW = NUM_SC * NUM_SUBCORES

@functools.lru_cache(maxsize=None)
def _sc_mesh():
    return plsc.VectorSubcoreMesh(
        core_axis_name="c", subcore_axis_name="s",
        num_cores=NUM_SC, num_subcores=NUM_SUBCORES)


def _ln(o, g, b):
    mu = jnp.mean(o, axis=-1, keepdims=True)
    var = jnp.mean((o - mu) * (o - mu), axis=-1, keepdims=True)
    return (o - mu) / jnp.sqrt(var + 1e-5) * g + b


def _enc_x_body(x_ref, w_ref, b_ref, o_ref):
    i = pl.program_id(0)
    rows = i * BN + lax.broadcasted_iota(jnp.int32, (BN, 1), 0)
    h = jnp.dot(x_ref[...], w_ref[...], preferred_element_type=jnp.float32)
    h = h + b_ref[...]
    o_ref[...] = jnp.where(rows < N, h, 0.0)


def _enc_x(x_pad, w, b, it=False):
    return pl.pallas_call(
        _enc_x_body,
        grid=(N_PAD // BN,),
        in_specs=[
            pl.BlockSpec((BN, DF), lambda i: (i, 0)),
            pl.BlockSpec((DF, L), lambda i: (0, 0)),
            pl.BlockSpec((1, L), lambda i: (0, 0)),
        ],
        out_specs=pl.BlockSpec((BN, L), lambda i: (i, 0)),
        out_shape=jax.ShapeDtypeStruct((N_PAD, L), jnp.float32),
        interpret=it,
    )(x_pad, w, b.reshape(1, L))


def _enc_e_body(a_ref, w_ref, b_ref, o_ref):
    o_ref[...] = (
        jnp.dot(a_ref[...], w_ref[...], preferred_element_type=jnp.float32)
        + b_ref[...]
    )


def _enc_e(ea, w, b, it=False):
    return pl.pallas_call(
        _enc_e_body,
        grid=(E // BE,),
        in_specs=[
            pl.BlockSpec((BE, DE), lambda i: (i, 0)),
            pl.BlockSpec((DE, L), lambda i: (0, 0)),
            pl.BlockSpec((1, L), lambda i: (0, 0)),
        ],
        out_specs=pl.BlockSpec((BE, L), lambda i: (i, 0)),
        out_shape=jax.ShapeDtypeStruct((E, L), jnp.float32),
        interpret=it,
    )(ea, w, b.reshape(1, L))


def _bdot(a, b):
    return jnp.dot(a, b, preferred_element_type=jnp.float32)


def _edge_mlp_body(e_ref, hs_ref, hd_ref, u_ref, e2g_ref, w1e_ref, w1s_ref,
                   w1d_ref, w1u_ref, b1_ref, w2_ref, b2_ref, g_ref, be_ref,
                   att_ref, en_ref, bm_ref):
    e = e_ref[...]
    cols = lax.broadcasted_iota(jnp.int32, (BE, G), 1)
    oh = (cols == e2g_ref[...]).astype(jnp.float32)
    ug = jnp.dot(oh, u_ref[...], preferred_element_type=jnp.float32)
    z = (
        _bdot(e, w1e_ref[...])
        + _bdot(hs_ref[...], w1s_ref[...])
        + _bdot(hd_ref[...], w1d_ref[...])
        + _bdot(ug, w1u_ref[...])
        + b1_ref[...]
    )
    z = jnp.maximum(z, 0.0)
    o = _bdot(z, w2_ref[...]) + b2_ref[...]
    e_new = e + _ln(o, g_ref[...], be_ref[...])
    en_ref[...] = e_new
    logits = jnp.sum(e_new * att_ref[...], axis=1, keepdims=True)
    bm_ref[...] = jnp.full((1, 1, L), jnp.max(logits), jnp.float32)


def _edge_mlp(e, hs, hd, u, e2g_col, sp, att, it=False):
    w1 = sp["W1"]
    return pl.pallas_call(
        _edge_mlp_body,
        grid=(E // BE,),
        in_specs=[pl.BlockSpec((BE, L), lambda i: (i, 0))] * 3
        + [
            pl.BlockSpec((G, L), lambda i: (0, 0)),
            pl.BlockSpec((BE, 1), lambda i: (i, 0)),
        ]
        + [pl.BlockSpec((L, H), lambda i: (0, 0))] * 4
        + [
            pl.BlockSpec((1, H), lambda i: (0, 0)),
            pl.BlockSpec((H, L), lambda i: (0, 0)),
            pl.BlockSpec((1, L), lambda i: (0, 0)),
            pl.BlockSpec((1, L), lambda i: (0, 0)),
            pl.BlockSpec((1, L), lambda i: (0, 0)),
            pl.BlockSpec((1, L), lambda i: (0, 0)),
        ],
        out_specs=[
            pl.BlockSpec((BE, L), lambda i: (i, 0)),
            pl.BlockSpec((1, 1, L), lambda i: (i, 0, 0)),
        ],
        out_shape=[
            jax.ShapeDtypeStruct((E, L), jnp.float32),
            jax.ShapeDtypeStruct((E // BE, 1, L), jnp.float32),
        ],
        interpret=it,
    )(
        e, hs, hd, u, e2g_col,
        w1[0:L], w1[L:2 * L], w1[2 * L:3 * L], w1[3 * L:4 * L],
        sp["b1"].reshape(1, H), sp["W2"], sp["b2"].reshape(1, L),
        sp["g"].reshape(1, L), sp["be"].reshape(1, L), att,
    )


def _edge_exp_body(en_ref, bm_ref, att_ref, ep_ref, p_ref):
    gmax = jnp.max(bm_ref[...])
    e = en_ref[...]
    logits = jnp.sum(e * att_ref[...], axis=1, keepdims=True)
    p = jnp.exp(logits - gmax)
    ep_ref[...] = e * p
    p_ref[...] = p


def _edge_exp(e_new, bm, att, it=False):
    return pl.pallas_call(
        _edge_exp_body,
        grid=(E // BE,),
        in_specs=[
            pl.BlockSpec((BE, L), lambda i: (i, 0)),
            pl.BlockSpec((E // BE, 1, L), lambda i: (0, 0, 0)),
            pl.BlockSpec((1, L), lambda i: (0, 0)),
        ],
        out_specs=[
            pl.BlockSpec((BE, L), lambda i: (i, 0)),
            pl.BlockSpec((BE, 1), lambda i: (i, 0)),
        ],
        out_shape=[
            jax.ShapeDtypeStruct((E, L), jnp.float32),
            jax.ShapeDtypeStruct((E, 1), jnp.float32),
        ],
        interpret=it,
    )(e_new, bm, att)


def _node_mlp_body(h_ref, acc_ref, s_ref, u_ref, n2g_ref, w1h_ref, w1a_ref,
                   w1u_ref, b1_ref, w2_ref, b2_ref, g_ref, be_ref, hn_ref):
    i = pl.program_id(0)
    rows = i * BN + lax.broadcasted_iota(jnp.int32, (BN, 1), 0)
    a = acc_ref[0] + acc_ref[1]
    agg = a / (s_ref[0] + s_ref[1] + 1e-16)
    h = h_ref[...]
    cols = lax.broadcasted_iota(jnp.int32, (BN, G), 1)
    oh = (cols == n2g_ref[...]).astype(jnp.float32)
    un = jnp.dot(oh, u_ref[...], preferred_element_type=jnp.float32)
    z = (
        _bdot(h, w1h_ref[...])
        + _bdot(agg, w1a_ref[...])
        + _bdot(un, w1u_ref[...])
        + b1_ref[...]
    )
    z = jnp.maximum(z, 0.0)
    o = _bdot(z, w2_ref[...]) + b2_ref[...]
    hn_ref[...] = jnp.where(rows < N, h + _ln(o, g_ref[...], be_ref[...]), 0.0)


def _node_mlp(h, acc2, s2, u, n2g_col, sp, it=False):
    w1 = sp["W1"]
    return pl.pallas_call(
        _node_mlp_body,
        grid=(N_PAD // BN,),
        in_specs=[
            pl.BlockSpec((BN, L), lambda i: (i, 0)),
            pl.BlockSpec((2, BN, L), lambda i: (0, i, 0)),
            pl.BlockSpec((2, BN, 1), lambda i: (0, i, 0)),
            pl.BlockSpec((G, L), lambda i: (0, 0)),
            pl.BlockSpec((BN, 1), lambda i: (i, 0)),
            pl.BlockSpec((L, H), lambda i: (0, 0)),
            pl.BlockSpec((L, H), lambda i: (0, 0)),
            pl.BlockSpec((L, H), lambda i: (0, 0)),
            pl.BlockSpec((1, H), lambda i: (0, 0)),
            pl.BlockSpec((H, L), lambda i: (0, 0)),
            pl.BlockSpec((1, L), lambda i: (0, 0)),
            pl.BlockSpec((1, L), lambda i: (0, 0)),
            pl.BlockSpec((1, L), lambda i: (0, 0)),
        ],
        out_specs=pl.BlockSpec((BN, L), lambda i: (i, 0)),
        out_shape=jax.ShapeDtypeStruct((N_PAD, L), jnp.float32),
        interpret=it,
    )(
        h, acc2, s2, u, n2g_col,
        w1[0:L], w1[L:2 * L], w1[2 * L:3 * L],
        sp["b1"].reshape(1, H), sp["W2"], sp["b2"].reshape(1, L),
        sp["g"].reshape(1, L), sp["be"].reshape(1, L),
    )


def _seg_mm_body(rows_ref, seg_ref, out_ref, *, bm):
    i = pl.program_id(0)
    cols = lax.broadcasted_iota(jnp.int32, (bm, G), 1)
    oh = (cols == seg_ref[...]).astype(jnp.float32)
    part = lax.dot_general(oh, rows_ref[...], (((0,), (0,)), ((), ())),
                           preferred_element_type=jnp.float32)

    @pl.when(i == 0)
    def _():
        out_ref[...] = part

    @pl.when(i > 0)
    def _():
        out_ref[...] += part


def _seg_mm(rows, seg_col, bm, it=False):
    m = rows.shape[0]
    return pl.pallas_call(
        functools.partial(_seg_mm_body, bm=bm),
        grid=(m // bm,),
        in_specs=[
            pl.BlockSpec((bm, L), lambda i: (i, 0)),
            pl.BlockSpec((bm, 1), lambda i: (i, 0)),
        ],
        out_specs=pl.BlockSpec((G, L), lambda i: (0, 0)),
        out_shape=jax.ShapeDtypeStruct((G, L), jnp.float32),
        compiler_params=pltpu.CompilerParams(
            dimension_semantics=("arbitrary",)),
        interpret=it,
    )(rows, seg_col)


def _glob_mlp_body(u_ref, ns_ref, es_ref, w1u_ref, w1n_ref, w1e_ref, b1_ref,
                   w2_ref, b2_ref, g_ref, be_ref, un_ref):
    u = u_ref[...]
    ns = ns_ref[...]
    es = es_ref[...]
    z = (
        jnp.dot(u, w1u_ref[...], preferred_element_type=jnp.float32)
        + jnp.dot(ns, w1n_ref[...], preferred_element_type=jnp.float32)
        + jnp.dot(es, w1e_ref[...], preferred_element_type=jnp.float32)
        + b1_ref[...]
    )
    z = jnp.maximum(z, 0.0)
    o = jnp.dot(z, w2_ref[...], preferred_element_type=jnp.float32) + b2_ref[...]
    un_ref[...] = u + _ln(o, g_ref[...], be_ref[...])


def _glob_mlp(u, ns, es, sp, it=False):
    w1 = sp["W1"]
    return pl.pallas_call(
        _glob_mlp_body,
        grid=(1,),
        in_specs=[
            pl.BlockSpec((G, L), lambda i: (0, 0)),
            pl.BlockSpec((G, L), lambda i: (0, 0)),
            pl.BlockSpec((G, L), lambda i: (0, 0)),
            pl.BlockSpec((L, H), lambda i: (0, 0)),
            pl.BlockSpec((L, H), lambda i: (0, 0)),
            pl.BlockSpec((L, H), lambda i: (0, 0)),
            pl.BlockSpec((1, H), lambda i: (0, 0)),
            pl.BlockSpec((H, L), lambda i: (0, 0)),
            pl.BlockSpec((1, L), lambda i: (0, 0)),
            pl.BlockSpec((1, L), lambda i: (0, 0)),
            pl.BlockSpec((1, L), lambda i: (0, 0)),
        ],
        out_specs=pl.BlockSpec((G, L), lambda i: (0, 0)),
        out_shape=jax.ShapeDtypeStruct((G, L), jnp.float32),
        interpret=it,
    )(
        u, ns, es,
        w1[0:L], w1[L:2 * L], w1[2 * L:3 * L],
        sp["b1"].reshape(1, H), sp["W2"], sp["b2"].reshape(1, L),
        sp["g"].reshape(1, L), sp["be"].reshape(1, L),
    )


def _proj_body(ns_ref, u_ref, w1_ref, b1_ref, w2_ref, b2_ref, z_ref):
    mol = ns_ref[...] + u_ref[...]
    t = jnp.maximum(
        jnp.dot(mol, w1_ref[...], preferred_element_type=jnp.float32)
        + b1_ref[...], 0.0)
    z_ref[...] = (
        jnp.dot(t, w2_ref[...], preferred_element_type=jnp.float32)
        + b2_ref[...]
    )


def _proj(ns, u, pr, it=False):
    return pl.pallas_call(
        _proj_body,
        grid=(1,),
        in_specs=[
            pl.BlockSpec((G, L), lambda i: (0, 0)),
            pl.BlockSpec((G, L), lambda i: (0, 0)),
            pl.BlockSpec((L, L), lambda i: (0, 0)),
            pl.BlockSpec((1, L), lambda i: (0, 0)),
            pl.BlockSpec((L, L), lambda i: (0, 0)),
            pl.BlockSpec((1, L), lambda i: (0, 0)),
        ],
        out_specs=pl.BlockSpec((G, L), lambda i: (0, 0)),
        out_shape=jax.ShapeDtypeStruct((G, L), jnp.float32),
        interpret=it,
    )(ns, u, pr["W1"], pr["b1"].reshape(1, L), pr["W2"],
      pr["b2"].reshape(1, L))


def _worker_id():
    return lax.axis_index("s") * NUM_SC + lax.axis_index("c")


def _edge_chunk_range(w):
    c0 = w * 40
    cnt = jnp.where(w < 31, 40, EC - 31 * 40)
    return c0, cnt


def _node_chunk_range(w):
    g0 = w * 2 + jnp.minimum(w, 16)
    cnt = 2 + jnp.where(w < 16, 1, 0)
    return g0, cnt


def _stage_idx(idx_hbm, idx_v, c0):
    pltpu.sync_copy(idx_hbm.at[pl.ds(c0, 40)], idx_v)


def _sc_e2g_body(n2g_ref, dst_ref, out_ref, n2g_v, idx_v, out_v, sem):
    del sem
    w = _worker_id()
    c0, cnt = _edge_chunk_range(w)
    pltpu.sync_copy(n2g_ref, n2g_v)
    _stage_idx(dst_ref, idx_v, c0)

    @pl.loop(0, cnt)
    def _(j):
        for k in range(8):
            idx16 = idx_v[j, pl.ds(16 * k, 16)]
            out_v[j, pl.ds(16 * k, 16)] = plsc.load_gather(n2g_v, [idx16])

    pltpu.sync_copy(out_v, out_ref.at[pl.ds(c0, 40)])


def _sc_e2g(n2g, dst2d):
    return pl.kernel(
        _sc_e2g_body,
        out_type=jax.ShapeDtypeStruct((EC_PAD, 128), jnp.int32),
        mesh=_sc_mesh(),
        compiler_params=pltpu.CompilerParams(needs_layout_passes=False),
        scratch_types=[
            pltpu.VMEM((N,), jnp.int32),
            pltpu.VMEM((40, 128), jnp.int32),
            pltpu.VMEM((40, 128), jnp.int32),
            pltpu.SemaphoreType.DMA,
        ],
    )(n2g, dst2d)


def _sc_gather_body(h_ref, src_ref, dst_ref, hs_ref, hd_ref, idx_v,
                    r0, r1, r2, r3, gs0, gs1, gs2, gs3, ss0, ss1, ss2, ss3):
    w = _worker_id()
    c0, cnt = _edge_chunk_range(w)
    bufs = (r0, r1, r2, r3)
    gsems = (gs0, gs1, gs2, gs3)
    ssems = (ss0, ss1, ss2, ss3)

    for idx_hbm, table, out in (
        (src_ref, h_ref, hs_ref),
        (dst_ref, h_ref, hd_ref),
    ):
        _stage_idx(idx_hbm, idx_v, c0)

        @pl.loop(0, cnt // 4)
        def _(grp, table=table, out=out):
            j0 = 4 * grp
            gds = [
                pltpu.async_copy(table.at[idx_v.at[j0 + b]], bufs[b],
                                 gsems[b])
                for b in range(4)
            ]
            sds = []
            for b in range(4):
                gds[b].wait()
                sds.append(pltpu.async_copy(
                    bufs[b], out.at[pl.ds((c0 + j0 + b) * 128, 128)],
                    ssems[b]))
            for b in range(4):
                sds[b].wait()

        @pl.loop(4 * (cnt // 4), cnt)
        def _(j, table=table, out=out):
            pltpu.async_copy(table.at[idx_v.at[j]], r0, gs0).wait()
            pltpu.sync_copy(r0, out.at[pl.ds((c0 + j) * 128, 128)])


def _sc_gather(h_pad, src2d, dst2d):
    return pl.kernel(
        _sc_gather_body,
        out_type=[
            jax.ShapeDtypeStruct((E, L), jnp.float32),
            jax.ShapeDtypeStruct((E, L), jnp.float32),
        ],
        mesh=_sc_mesh(),
        scratch_types=[
            pltpu.VMEM((40, 128), jnp.int32),
            pltpu.VMEM((128, L), jnp.float32),
            pltpu.VMEM((128, L), jnp.float32),
            pltpu.VMEM((128, L), jnp.float32),
            pltpu.VMEM((128, L), jnp.float32),
        ] + [pltpu.SemaphoreType.DMA] * 8,
    )(h_pad, src2d, dst2d)


ACC_ROWS = N_PAD + 128
SROW = 10160


def _edge_chunk_range64(w):
    c0 = w * 80
    cnt = jnp.where(w < 31, 80, (E // 64) - 31 * 80)
    return c0, cnt


def _sc_scatter_edges_body(ep_ref, dst_ref, p_ref, zacc_ref,
                           acc_out, acc_sh, idxd_v, p_v,
                           s_local, idr_v, r0, r1, gs0, gs1, ss0, ss1):
    c = lax.axis_index("c")
    s = lax.axis_index("s")
    w = _worker_id()
    c0, cnt = _edge_chunk_range64(w)

    pltpu.sync_copy(zacc_ref.at[pl.ds(s * (ACC_ROWS // 16), ACC_ROWS // 16)],
                    acc_sh.at[pl.ds(s * (ACC_ROWS // 16), ACC_ROWS // 16)])
    pltpu.sync_copy(zacc_ref.at[pl.ds(0, NPC)], s_local)

    iota16 = lax.broadcasted_iota(jnp.int32, (16,), 0)
    for k in range(5):
        idr_v[0, pl.ds(16 * k, 16)] = iota16 + (SROW + 16 * k)

    plsc.subcore_barrier()

    pltpu.sync_copy(dst_ref.at[pl.ds(c0, 80)], idxd_v)
    pltpu.sync_copy(p_ref.at[pl.ds(c0, 80)], p_v)

    @pl.loop(0, cnt // 2)
    def _(grp):
        j0 = 2 * grp
        j1 = j0 + 1
        g0 = pltpu.async_copy(ep_ref.at[pl.ds((c0 + j0) * 64, 64)], r0, gs0)
        g1 = pltpu.async_copy(ep_ref.at[pl.ds((c0 + j1) * 64, 64)], r1, gs1)
        g0.wait()
        s0 = pltpu.async_copy(r0, acc_sh.at[idxd_v.at[j0]], ss0, add=True)
        for k in range(4):
            d16 = idxd_v[j0, pl.ds(16 * k, 16)]
            p16 = p_v[j0, pl.ds(16 * k, 16)]
            plsc.addupdate_scatter(
                s_local, [lax.shift_right_logical(d16, 7), d16 & 127], p16)
        g1.wait()
        s1 = pltpu.async_copy(r1, acc_sh.at[idxd_v.at[j1]], ss1, add=True)
        for k in range(4):
            d16 = idxd_v[j1, pl.ds(16 * k, 16)]
            p16 = p_v[j1, pl.ds(16 * k, 16)]
            plsc.addupdate_scatter(
                s_local, [lax.shift_right_logical(d16, 7), d16 & 127], p16)
        s0.wait()
        s1.wait()

    pltpu.sync_copy(s_local, acc_sh.at[idr_v.at[0]], add=True)

    plsc.subcore_barrier()

    @pl.when(s == 0)
    def _():
        pltpu.sync_copy(acc_sh, acc_out.at[c])


def _sc_scatter_edges(ep, dst64, p64, zacc):
    return pl.kernel(
        _sc_scatter_edges_body,
        out_type=jax.ShapeDtypeStruct((NUM_SC, ACC_ROWS, L), jnp.float32),
        mesh=_sc_mesh(),
        compiler_params=pltpu.CompilerParams(needs_layout_passes=False),
        scratch_types=[
            pltpu.VMEM_SHARED((ACC_ROWS, L), jnp.float32),
            pltpu.VMEM((80, 64), jnp.int32),
            pltpu.VMEM((80, 64), jnp.float32),
            pltpu.VMEM((NPC, 128), jnp.float32),
            pltpu.VMEM((1, 80), jnp.int32),
            pltpu.VMEM((64, L), jnp.float32),
            pltpu.VMEM((64, L), jnp.float32),
        ] + [pltpu.SemaphoreType.DMA] * 4,
    )(ep, dst64, p64, zacc)


def kernel(x, edge_index, edge_attr, node2graph, params):
    pad_e = EC_PAD * 128 - E
    src2d = jnp.pad(edge_index[0], (0, pad_e)).reshape(EC_PAD, 128)
    dst2d = jnp.pad(edge_index[1], (0, pad_e)).reshape(EC_PAD, 128)
    dst64 = jnp.pad(edge_index[1], (0, pad_e)).reshape(EC_PAD * 2, 64)
    n2g_col = jnp.pad(node2graph, (0, N_PAD - N)).reshape(N_PAD, 1)
    x_pad = jnp.pad(x, ((0, N_PAD - N), (0, 0)))
    zacc = jnp.zeros((ACC_ROWS, L), jnp.float32)

    e2g2d = _sc_e2g(node2graph, dst2d)
    e2g_col = e2g2d.reshape(EC_PAD * 128)[:E].reshape(E, 1)

    h = _enc_x(x_pad, params["enc_x"]["W"], params["enc_x"]["b"])
    e = _enc_e(edge_attr, params["enc_e"]["W"], params["enc_e"]["b"])
    u = jnp.zeros((G, L), jnp.float32)

    for sp in params["steps"]:
        att = sp["att"].reshape(1, L)
        hs, hd = _sc_gather(h, src2d, dst2d)
        e, bm = _edge_mlp(e, hs, hd, u, e2g_col, sp["edge"], att)
        ep, pcol = _edge_exp(e, bm, att)
        p64 = jnp.pad(pcol.reshape(E), (0, pad_e)).reshape(EC_PAD * 2, 64)
        acc2 = _sc_scatter_edges(ep, dst64, p64, zacc)
        s2 = acc2[:, SROW:SROW + NPC, :].reshape(NUM_SC, N_PAD)[:, :, None]
        h = _node_mlp(h, acc2, s2, u, n2g_col, sp["node"])
        ns = _seg_mm(h, n2g_col, BN)
        es = _seg_mm(e, e2g_col, BE)
        u = _glob_mlp(u, ns, es, sp["glob"])

    return _proj(ns, u, params["proj"])


# speedup vs baseline: 1.0509x; 1.0021x over previous
import functools

import jax
import jax.numpy as jnp
from jax import lax
from jax.experimental import pallas as pl
from jax.experimental.pallas import tpu as pltpu
from jax.experimental.pallas import tpu_sc as plsc

N = 10000
E = 160000
DF = 128
DE = 16
G = 256
L = 128
H = 512

N_PAD = 10240
EC = E // 128
EC_PAD = 1280
NPC = N_PAD // 128

BE = 3200
BN = 2048

NUM_SC = 2
NUM_SUBCORES = 16
NW = NUM_SC * NUM_SUBCORES

@functools.lru_cache(maxsize=None)
def _sc_mesh():
    return plsc.VectorSubcoreMesh(
        core_axis_name="c", subcore_axis_name="s",
        num_cores=NUM_SC, num_subcores=NUM_SUBCORES)


def _ln(o, g, b):
    mu = jnp.mean(o, axis=-1, keepdims=True)
    var = jnp.mean((o - mu) * (o - mu), axis=-1, keepdims=True)
    return (o - mu) / jnp.sqrt(var + 1e-5) * g + b


def _enc_x_body(x_ref, w_ref, b_ref, o_ref):
    i = pl.program_id(0)
    rows = i * BN + lax.broadcasted_iota(jnp.int32, (BN, 1), 0)
    h = jnp.dot(x_ref[...], w_ref[...], preferred_element_type=jnp.float32)
    h = h + b_ref[...]
    o_ref[...] = jnp.where(rows < N, h, 0.0)


def _enc_x(x_pad, w, b, it=False):
    return pl.pallas_call(
        _enc_x_body,
        grid=(N_PAD // BN,),
        in_specs=[
            pl.BlockSpec((BN, DF), lambda i: (i, 0)),
            pl.BlockSpec((DF, L), lambda i: (0, 0)),
            pl.BlockSpec((1, L), lambda i: (0, 0)),
        ],
        out_specs=pl.BlockSpec((BN, L), lambda i: (i, 0)),
        out_shape=jax.ShapeDtypeStruct((N_PAD, L), jnp.float32),
        interpret=it,
    )(x_pad, w, b.reshape(1, L))


def _enc_e_body(a_ref, w_ref, b_ref, o_ref):
    o_ref[...] = (
        jnp.dot(a_ref[...], w_ref[...], preferred_element_type=jnp.float32)
        + b_ref[...]
    )


def _enc_e(ea, w, b, it=False):
    return pl.pallas_call(
        _enc_e_body,
        grid=(E // BE,),
        in_specs=[
            pl.BlockSpec((BE, DE), lambda i: (i, 0)),
            pl.BlockSpec((DE, L), lambda i: (0, 0)),
            pl.BlockSpec((1, L), lambda i: (0, 0)),
        ],
        out_specs=pl.BlockSpec((BE, L), lambda i: (i, 0)),
        out_shape=jax.ShapeDtypeStruct((E, L), jnp.float32),
        interpret=it,
    )(ea, w, b.reshape(1, L))


def _bdot(a, b):
    return jnp.dot(a, b, preferred_element_type=jnp.float32)


def _edge_mlp_body(e_ref, hs_ref, hd_ref, u_ref, e2g_ref, w1e_ref, w1s_ref,
                   w1d_ref, w1u_ref, b1_ref, w2_ref, b2_ref, g_ref, be_ref,
                   att_ref, en_ref, bm_ref):
    e = e_ref[...]
    cols = lax.broadcasted_iota(jnp.int32, (BE, G), 1)
    oh = (cols == e2g_ref[...]).astype(jnp.float32)
    ug = jnp.dot(oh, u_ref[...], preferred_element_type=jnp.float32)
    z = (
        _bdot(e, w1e_ref[...])
        + _bdot(hs_ref[...], w1s_ref[...])
        + _bdot(hd_ref[...], w1d_ref[...])
        + _bdot(ug, w1u_ref[...])
        + b1_ref[...]
    )
    z = jnp.maximum(z, 0.0)
    o = _bdot(z, w2_ref[...]) + b2_ref[...]
    e_new = e + _ln(o, g_ref[...], be_ref[...])
    en_ref[...] = e_new
    logits = jnp.sum(e_new * att_ref[...], axis=1, keepdims=True)
    bm_ref[...] = jnp.full((1, 1, L), jnp.max(logits), jnp.float32)


def _edge_mlp(e, hs, hd, u, e2g_col, sp, att, it=False):
    w1 = sp["W1"]
    return pl.pallas_call(
        _edge_mlp_body,
        grid=(E // BE,),
        in_specs=[pl.BlockSpec((BE, L), lambda i: (i, 0))] * 3
        + [
            pl.BlockSpec((G, L), lambda i: (0, 0)),
            pl.BlockSpec((BE, 1), lambda i: (i, 0)),
        ]
        + [pl.BlockSpec((L, H), lambda i: (0, 0))] * 4
        + [
            pl.BlockSpec((1, H), lambda i: (0, 0)),
            pl.BlockSpec((H, L), lambda i: (0, 0)),
            pl.BlockSpec((1, L), lambda i: (0, 0)),
            pl.BlockSpec((1, L), lambda i: (0, 0)),
            pl.BlockSpec((1, L), lambda i: (0, 0)),
            pl.BlockSpec((1, L), lambda i: (0, 0)),
        ],
        out_specs=[
            pl.BlockSpec((BE, L), lambda i: (i, 0)),
            pl.BlockSpec((1, 1, L), lambda i: (i, 0, 0)),
        ],
        out_shape=[
            jax.ShapeDtypeStruct((E, L), jnp.float32),
            jax.ShapeDtypeStruct((E // BE, 1, L), jnp.float32),
        ],
        interpret=it,
    )(
        e, hs, hd, u, e2g_col,
        w1[0:L], w1[L:2 * L], w1[2 * L:3 * L], w1[3 * L:4 * L],
        sp["b1"].reshape(1, H), sp["W2"], sp["b2"].reshape(1, L),
        sp["g"].reshape(1, L), sp["be"].reshape(1, L), att,
    )


def _edge_exp_body(en_ref, bm_ref, att_ref, e2g_ref, ep_ref, p_ref, es_ref):
    i = pl.program_id(0)
    gmax = jnp.max(bm_ref[...])
    e = en_ref[...]
    logits = jnp.sum(e * att_ref[...], axis=1, keepdims=True)
    p = jnp.exp(logits - gmax)
    ep_ref[...] = e * p
    p_ref[...] = p
    cols = lax.broadcasted_iota(jnp.int32, (BE, G), 1)
    oh = (cols == e2g_ref[...]).astype(jnp.float32)
    es_part = lax.dot_general(oh, e, (((0,), (0,)), ((), ())),
                              preferred_element_type=jnp.float32)

    @pl.when(i == 0)
    def _():
        es_ref[...] = es_part

    @pl.when(i > 0)
    def _():
        es_ref[...] += es_part


def _edge_exp(e_new, bm, att, e2g_col, it=False):
    return pl.pallas_call(
        _edge_exp_body,
        grid=(E // BE,),
        in_specs=[
            pl.BlockSpec((BE, L), lambda i: (i, 0)),
            pl.BlockSpec((E // BE, 1, L), lambda i: (0, 0, 0)),
            pl.BlockSpec((1, L), lambda i: (0, 0)),
            pl.BlockSpec((BE, 1), lambda i: (i, 0)),
        ],
        out_specs=[
            pl.BlockSpec((BE, L), lambda i: (i, 0)),
            pl.BlockSpec((BE, 1), lambda i: (i, 0)),
            pl.BlockSpec((G, L), lambda i: (0, 0)),
        ],
        out_shape=[
            jax.ShapeDtypeStruct((E, L), jnp.float32),
            jax.ShapeDtypeStruct((E, 1), jnp.float32),
            jax.ShapeDtypeStruct((G, L), jnp.float32),
        ],
        interpret=it,
    )(e_new, bm, att, e2g_col)


def _node_mlp_body(h_ref, acc_ref, s_ref, u_ref, n2g_ref, w1h_ref, w1a_ref,
                   w1u_ref, b1_ref, w2_ref, b2_ref, g_ref, be_ref, hn_ref):
    i = pl.program_id(0)
    rows = i * BN + lax.broadcasted_iota(jnp.int32, (BN, 1), 0)
    a = acc_ref[0] + acc_ref[1]
    agg = a / (s_ref[0] + s_ref[1] + 1e-16)
    h = h_ref[...]
    cols = lax.broadcasted_iota(jnp.int32, (BN, G), 1)
    oh = (cols == n2g_ref[...]).astype(jnp.float32)
    un = jnp.dot(oh, u_ref[...], preferred_element_type=jnp.float32)
    z = (
        _bdot(h, w1h_ref[...])
        + _bdot(agg, w1a_ref[...])
        + _bdot(un, w1u_ref[...])
        + b1_ref[...]
    )
    z = jnp.maximum(z, 0.0)
    o = _bdot(z, w2_ref[...]) + b2_ref[...]
    hn_ref[...] = jnp.where(rows < N, h + _ln(o, g_ref[...], be_ref[...]), 0.0)


def _node_mlp(h, acc2, s2, u, n2g_col, sp, it=False):
    w1 = sp["W1"]
    return pl.pallas_call(
        _node_mlp_body,
        grid=(N_PAD // BN,),
        in_specs=[
            pl.BlockSpec((BN, L), lambda i: (i, 0)),
            pl.BlockSpec((2, BN, L), lambda i: (0, i, 0)),
            pl.BlockSpec((2, BN, 1), lambda i: (0, i, 0)),
            pl.BlockSpec((G, L), lambda i: (0, 0)),
            pl.BlockSpec((BN, 1), lambda i: (i, 0)),
            pl.BlockSpec((L, H), lambda i: (0, 0)),
            pl.BlockSpec((L, H), lambda i: (0, 0)),
            pl.BlockSpec((L, H), lambda i: (0, 0)),
            pl.BlockSpec((1, H), lambda i: (0, 0)),
            pl.BlockSpec((H, L), lambda i: (0, 0)),
            pl.BlockSpec((1, L), lambda i: (0, 0)),
            pl.BlockSpec((1, L), lambda i: (0, 0)),
            pl.BlockSpec((1, L), lambda i: (0, 0)),
        ],
        out_specs=pl.BlockSpec((BN, L), lambda i: (i, 0)),
        out_shape=jax.ShapeDtypeStruct((N_PAD, L), jnp.float32),
        interpret=it,
    )(
        h, acc2, s2, u, n2g_col,
        w1[0:L], w1[L:2 * L], w1[2 * L:3 * L],
        sp["b1"].reshape(1, H), sp["W2"], sp["b2"].reshape(1, L),
        sp["g"].reshape(1, L), sp["be"].reshape(1, L),
    )


def _seg_mm_body(rows_ref, seg_ref, out_ref, *, bm):
    i = pl.program_id(0)
    cols = lax.broadcasted_iota(jnp.int32, (bm, G), 1)
    oh = (cols == seg_ref[...]).astype(jnp.float32)
    part = lax.dot_general(oh, rows_ref[...], (((0,), (0,)), ((), ())),
                           preferred_element_type=jnp.float32)

    @pl.when(i == 0)
    def _():
        out_ref[...] = part

    @pl.when(i > 0)
    def _():
        out_ref[...] += part


def _seg_mm(rows, seg_col, bm, it=False):
    m = rows.shape[0]
    return pl.pallas_call(
        functools.partial(_seg_mm_body, bm=bm),
        grid=(m // bm,),
        in_specs=[
            pl.BlockSpec((bm, L), lambda i: (i, 0)),
            pl.BlockSpec((bm, 1), lambda i: (i, 0)),
        ],
        out_specs=pl.BlockSpec((G, L), lambda i: (0, 0)),
        out_shape=jax.ShapeDtypeStruct((G, L), jnp.float32),
        compiler_params=pltpu.CompilerParams(
            dimension_semantics=("arbitrary",)),
        interpret=it,
    )(rows, seg_col)


def _glob_mlp_body(u_ref, ns_ref, es_ref, w1u_ref, w1n_ref, w1e_ref, b1_ref,
                   w2_ref, b2_ref, g_ref, be_ref, un_ref):
    u = u_ref[...]
    ns = ns_ref[...]
    es = es_ref[...]
    z = (
        jnp.dot(u, w1u_ref[...], preferred_element_type=jnp.float32)
        + jnp.dot(ns, w1n_ref[...], preferred_element_type=jnp.float32)
        + jnp.dot(es, w1e_ref[...], preferred_element_type=jnp.float32)
        + b1_ref[...]
    )
    z = jnp.maximum(z, 0.0)
    o = jnp.dot(z, w2_ref[...], preferred_element_type=jnp.float32) + b2_ref[...]
    un_ref[...] = u + _ln(o, g_ref[...], be_ref[...])


def _glob_mlp(u, ns, es, sp, it=False):
    w1 = sp["W1"]
    return pl.pallas_call(
        _glob_mlp_body,
        grid=(1,),
        in_specs=[
            pl.BlockSpec((G, L), lambda i: (0, 0)),
            pl.BlockSpec((G, L), lambda i: (0, 0)),
            pl.BlockSpec((G, L), lambda i: (0, 0)),
            pl.BlockSpec((L, H), lambda i: (0, 0)),
            pl.BlockSpec((L, H), lambda i: (0, 0)),
            pl.BlockSpec((L, H), lambda i: (0, 0)),
            pl.BlockSpec((1, H), lambda i: (0, 0)),
            pl.BlockSpec((H, L), lambda i: (0, 0)),
            pl.BlockSpec((1, L), lambda i: (0, 0)),
            pl.BlockSpec((1, L), lambda i: (0, 0)),
            pl.BlockSpec((1, L), lambda i: (0, 0)),
        ],
        out_specs=pl.BlockSpec((G, L), lambda i: (0, 0)),
        out_shape=jax.ShapeDtypeStruct((G, L), jnp.float32),
        interpret=it,
    )(
        u, ns, es,
        w1[0:L], w1[L:2 * L], w1[2 * L:3 * L],
        sp["b1"].reshape(1, H), sp["W2"], sp["b2"].reshape(1, L),
        sp["g"].reshape(1, L), sp["be"].reshape(1, L),
    )


def _proj_body(ns_ref, u_ref, w1_ref, b1_ref, w2_ref, b2_ref, z_ref):
    mol = ns_ref[...] + u_ref[...]
    t = jnp.maximum(
        jnp.dot(mol, w1_ref[...], preferred_element_type=jnp.float32)
        + b1_ref[...], 0.0)
    z_ref[...] = (
        jnp.dot(t, w2_ref[...], preferred_element_type=jnp.float32)
        + b2_ref[...]
    )


def _proj(ns, u, pr, it=False):
    return pl.pallas_call(
        _proj_body,
        grid=(1,),
        in_specs=[
            pl.BlockSpec((G, L), lambda i: (0, 0)),
            pl.BlockSpec((G, L), lambda i: (0, 0)),
            pl.BlockSpec((L, L), lambda i: (0, 0)),
            pl.BlockSpec((1, L), lambda i: (0, 0)),
            pl.BlockSpec((L, L), lambda i: (0, 0)),
            pl.BlockSpec((1, L), lambda i: (0, 0)),
        ],
        out_specs=pl.BlockSpec((G, L), lambda i: (0, 0)),
        out_shape=jax.ShapeDtypeStruct((G, L), jnp.float32),
        interpret=it,
    )(ns, u, pr["W1"], pr["b1"].reshape(1, L), pr["W2"],
      pr["b2"].reshape(1, L))


def _worker_id():
    return lax.axis_index("s") * NUM_SC + lax.axis_index("c")


def _edge_chunk_range(w):
    c0 = w * 40
    cnt = jnp.where(w < 31, 40, EC - 31 * 40)
    return c0, cnt


def _node_chunk_range(w):
    g0 = w * 2 + jnp.minimum(w, 16)
    cnt = 2 + jnp.where(w < 16, 1, 0)
    return g0, cnt


def _stage_idx(idx_hbm, idx_v, c0):
    pltpu.sync_copy(idx_hbm.at[pl.ds(c0, 40)], idx_v)


def _sc_e2g_body(n2g_ref, dst_ref, out_ref, n2g_v, idx_v, out_v, sem):
    del sem
    w = _worker_id()
    c0, cnt = _edge_chunk_range(w)
    pltpu.sync_copy(n2g_ref, n2g_v)
    _stage_idx(dst_ref, idx_v, c0)

    @pl.loop(0, cnt)
    def _(j):
        for k in range(8):
            idx16 = idx_v[j, pl.ds(16 * k, 16)]
            out_v[j, pl.ds(16 * k, 16)] = plsc.load_gather(n2g_v, [idx16])

    pltpu.sync_copy(out_v, out_ref.at[pl.ds(c0, 40)])


def _sc_e2g(n2g, dst2d):
    return pl.kernel(
        _sc_e2g_body,
        out_type=jax.ShapeDtypeStruct((EC_PAD, 128), jnp.int32),
        mesh=_sc_mesh(),
        compiler_params=pltpu.CompilerParams(needs_layout_passes=False),
        scratch_types=[
            pltpu.VMEM((N,), jnp.int32),
            pltpu.VMEM((40, 128), jnp.int32),
            pltpu.VMEM((40, 128), jnp.int32),
            pltpu.SemaphoreType.DMA,
        ],
    )(n2g, dst2d)


def _sc_gather_body(h_ref, src_ref, dst_ref, hs_ref, hd_ref, idx_v,
                    r0, r1, r2, r3, gs0, gs1, gs2, gs3, ss0, ss1, ss2, ss3):
    w = _worker_id()
    c0, cnt = _edge_chunk_range(w)
    bufs = (r0, r1, r2, r3)
    gsems = (gs0, gs1, gs2, gs3)
    ssems = (ss0, ss1, ss2, ss3)

    for idx_hbm, table, out in (
        (src_ref, h_ref, hs_ref),
        (dst_ref, h_ref, hd_ref),
    ):
        _stage_idx(idx_hbm, idx_v, c0)

        @pl.loop(0, cnt // 4)
        def _(grp, table=table, out=out):
            j0 = 4 * grp
            gds = [
                pltpu.async_copy(table.at[idx_v.at[j0 + b]], bufs[b],
                                 gsems[b])
                for b in range(4)
            ]
            sds = []
            for b in range(4):
                gds[b].wait()
                sds.append(pltpu.async_copy(
                    bufs[b], out.at[pl.ds((c0 + j0 + b) * 128, 128)],
                    ssems[b]))
            for b in range(4):
                sds[b].wait()

        @pl.loop(4 * (cnt // 4), cnt)
        def _(j, table=table, out=out):
            pltpu.async_copy(table.at[idx_v.at[j]], r0, gs0).wait()
            pltpu.sync_copy(r0, out.at[pl.ds((c0 + j) * 128, 128)])


def _sc_gather(h_pad, src2d, dst2d):
    return pl.kernel(
        _sc_gather_body,
        out_type=[
            jax.ShapeDtypeStruct((E, L), jnp.float32),
            jax.ShapeDtypeStruct((E, L), jnp.float32),
        ],
        mesh=_sc_mesh(),
        scratch_types=[
            pltpu.VMEM((40, 128), jnp.int32),
            pltpu.VMEM((128, L), jnp.float32),
            pltpu.VMEM((128, L), jnp.float32),
            pltpu.VMEM((128, L), jnp.float32),
            pltpu.VMEM((128, L), jnp.float32),
        ] + [pltpu.SemaphoreType.DMA] * 8,
    )(h_pad, src2d, dst2d)


ACC_ROWS = N_PAD + 128
SROW = 10160


def _edge_chunk_range64(w):
    c0 = w * 80
    cnt = jnp.where(w < 31, 80, (E // 64) - 31 * 80)
    return c0, cnt


def _sc_scatter_edges_body(ep_ref, dst_ref, p_ref, zacc_ref,
                           acc_out, acc_sh, idxd_v, p_v,
                           s_local, idr_v, r0, r1, gs0, gs1, ss0, ss1):
    c = lax.axis_index("c")
    s = lax.axis_index("s")
    w = _worker_id()
    c0, cnt = _edge_chunk_range64(w)

    pltpu.sync_copy(zacc_ref.at[pl.ds(s * (ACC_ROWS // 16), ACC_ROWS // 16)],
                    acc_sh.at[pl.ds(s * (ACC_ROWS // 16), ACC_ROWS // 16)])
    pltpu.sync_copy(zacc_ref.at[pl.ds(0, NPC)], s_local)

    iota16 = lax.broadcasted_iota(jnp.int32, (16,), 0)
    for k in range(5):
        idr_v[0, pl.ds(16 * k, 16)] = iota16 + (SROW + 16 * k)

    plsc.subcore_barrier()

    pltpu.sync_copy(dst_ref.at[pl.ds(c0, 80)], idxd_v)
    pltpu.sync_copy(p_ref.at[pl.ds(c0, 80)], p_v)

    @pl.loop(0, cnt // 2)
    def _(grp):
        j0 = 2 * grp
        j1 = j0 + 1
        g0 = pltpu.async_copy(ep_ref.at[pl.ds((c0 + j0) * 64, 64)], r0, gs0)
        g1 = pltpu.async_copy(ep_ref.at[pl.ds((c0 + j1) * 64, 64)], r1, gs1)
        g0.wait()
        s0 = pltpu.async_copy(r0, acc_sh.at[idxd_v.at[j0]], ss0, add=True)
        for k in range(4):
            d16 = idxd_v[j0, pl.ds(16 * k, 16)]
            p16 = p_v[j0, pl.ds(16 * k, 16)]
            plsc.addupdate_scatter(
                s_local, [lax.shift_right_logical(d16, 7), d16 & 127], p16)
        g1.wait()
        s1 = pltpu.async_copy(r1, acc_sh.at[idxd_v.at[j1]], ss1, add=True)
        for k in range(4):
            d16 = idxd_v[j1, pl.ds(16 * k, 16)]
            p16 = p_v[j1, pl.ds(16 * k, 16)]
            plsc.addupdate_scatter(
                s_local, [lax.shift_right_logical(d16, 7), d16 & 127], p16)
        s0.wait()
        s1.wait()

    pltpu.sync_copy(s_local, acc_sh.at[idr_v.at[0]], add=True)

    plsc.subcore_barrier()

    @pl.when(s == 0)
    def _():
        pltpu.sync_copy(acc_sh, acc_out.at[c])


def _sc_scatter_edges(ep, dst64, p64, zacc):
    return pl.kernel(
        _sc_scatter_edges_body,
        out_type=jax.ShapeDtypeStruct((NUM_SC, ACC_ROWS, L), jnp.float32),
        mesh=_sc_mesh(),
        compiler_params=pltpu.CompilerParams(needs_layout_passes=False),
        scratch_types=[
            pltpu.VMEM_SHARED((ACC_ROWS, L), jnp.float32),
            pltpu.VMEM((80, 64), jnp.int32),
            pltpu.VMEM((80, 64), jnp.float32),
            pltpu.VMEM((NPC, 128), jnp.float32),
            pltpu.VMEM((1, 80), jnp.int32),
            pltpu.VMEM((64, L), jnp.float32),
            pltpu.VMEM((64, L), jnp.float32),
        ] + [pltpu.SemaphoreType.DMA] * 4,
    )(ep, dst64, p64, zacc)


def kernel(x, edge_index, edge_attr, node2graph, params):
    pad_e = EC_PAD * 128 - E
    src2d = jnp.pad(edge_index[0], (0, pad_e)).reshape(EC_PAD, 128)
    dst2d = jnp.pad(edge_index[1], (0, pad_e)).reshape(EC_PAD, 128)
    dst64 = jnp.pad(edge_index[1], (0, pad_e)).reshape(EC_PAD * 2, 64)
    n2g_col = jnp.pad(node2graph, (0, N_PAD - N)).reshape(N_PAD, 1)
    x_pad = jnp.pad(x, ((0, N_PAD - N), (0, 0)))
    zacc = jnp.zeros((ACC_ROWS, L), jnp.float32)

    e2g2d = _sc_e2g(node2graph, dst2d)
    e2g_col = e2g2d.reshape(EC_PAD * 128)[:E].reshape(E, 1)

    h = _enc_x(x_pad, params["enc_x"]["W"], params["enc_x"]["b"])
    e = _enc_e(edge_attr, params["enc_e"]["W"], params["enc_e"]["b"])
    u = jnp.zeros((G, L), jnp.float32)

    for sp in params["steps"]:
        att = sp["att"].reshape(1, L)
        hs, hd = _sc_gather(h, src2d, dst2d)
        e, bm = _edge_mlp(e, hs, hd, u, e2g_col, sp["edge"], att)
        ep, pcol, es = _edge_exp(e, bm, att, e2g_col)
        p64 = jnp.pad(pcol.reshape(E), (0, pad_e)).reshape(EC_PAD * 2, 64)
        acc2 = _sc_scatter_edges(ep, dst64, p64, zacc)
        s2 = acc2[:, SROW:SROW + NPC, :].reshape(NUM_SC, N_PAD)[:, :, None]
        h = _node_mlp(h, acc2, s2, u, n2g_col, sp["node"])
        ns = _seg_mm(h, n2g_col, BN)
        u = _glob_mlp(u, ns, es, sp["glob"])

    return _proj(ns, u, params["proj"])
